# baseline (speedup 1.0000x reference)
import jax
import jax.numpy as jnp
from jax.experimental import pallas as pl
from jax.experimental.pallas import tpu as pltpu

K = 16
NUM_SAMPLES = 2
TAU = 1.0


def _dknn_block(s_ref, u_ref, out_ref):
    scores = s_ref[...]
    u = u_ref[0]
    p = scores + (-jnp.log(-jnp.log(u)))
    qb, n = p.shape
    d16 = (p[:, :, None] - p[:, None, :]).astype(jnp.bfloat16)
    d16 = jnp.abs(d16).reshape(qb * n, n)
    ones16 = jnp.ones((n, 1), dtype=jnp.bfloat16)
    r = jnp.dot(d16, ones16,
                preferred_element_type=jnp.float32).reshape(qb, n)
    i_idx = jax.lax.broadcasted_iota(jnp.int32, (K, n), 0).astype(p.dtype)
    c = (n - 1.0) - 2.0 * i_idx
    logits = c[None] * p[:, None, :] - r[:, None, :]
    m = jnp.max(logits, axis=-1, keepdims=True)
    e = jnp.exp(logits - m)
    probs = e / jnp.sum(e, axis=-1, keepdims=True)
    out_ref[0] = jnp.sum(probs, axis=1)


@jax.jit
def kernel(query, neighbors):
    Q, D = query.shape
    N, _ = neighbors.shape
    QB = 64
    diffs = query[:, None, :] - neighbors[None, :, :]
    squared_diffs = diffs ** 2
    l2_norms = squared_diffs.sum(axis=2)
    scores = -l2_norms
    gkey = jax.random.key(1234)
    u = jax.random.uniform(gkey, (NUM_SAMPLES, Q, N),
                           dtype=scores.dtype, minval=1e-8, maxval=1.0 - 1e-8)
    out = pl.pallas_call(
        _dknn_block,
        grid=(NUM_SAMPLES, Q // QB),
        in_specs=[
            pl.BlockSpec((QB, N), lambda s, qb: (qb, 0)),
            pl.BlockSpec((1, QB, N), lambda s, qb: (s, qb, 0)),
        ],
        out_specs=pl.BlockSpec((1, QB, N), lambda s, qb: (s, qb, 0)),
        out_shape=jax.ShapeDtypeStruct((NUM_SAMPLES, Q, N), query.dtype),
        compiler_params=pltpu.CompilerParams(
            dimension_semantics=("parallel", "parallel")),
    )(scores, u)
    return out


# speedup vs baseline: 1.0112x; 1.0112x over previous
import jax
import jax.numpy as jnp
from jax.experimental import pallas as pl
from jax.experimental.pallas import tpu as pltpu

K = 16
NUM_SAMPLES = 2
TAU = 1.0


def _dknn_block(p_ref, out_ref):
    p = p_ref[0]
    qb, n = p.shape
    d16 = (p[:, :, None] - p[:, None, :]).astype(jnp.bfloat16)
    d16 = jnp.abs(d16).reshape(qb * n, n)
    ones16 = jnp.ones((n, 1), dtype=jnp.bfloat16)
    r = jnp.dot(d16, ones16,
                preferred_element_type=jnp.float32).reshape(qb, n)
    i_idx = jax.lax.broadcasted_iota(jnp.int32, (K, n), 0).astype(p.dtype)
    c = (n - 1.0) - 2.0 * i_idx
    logits = c[None] * p[:, None, :] - r[:, None, :]
    m = jnp.max(logits, axis=-1, keepdims=True)
    e = jnp.exp(logits - m)
    probs = e / jnp.sum(e, axis=-1, keepdims=True)
    out_ref[0] = jnp.sum(probs, axis=1)


@jax.jit
def kernel(query, neighbors):
    Q, D = query.shape
    N, _ = neighbors.shape
    QB = 64
    diffs = query[:, None, :] - neighbors[None, :, :]
    squared_diffs = diffs ** 2
    l2_norms = squared_diffs.sum(axis=2)
    scores = -l2_norms
    gkey = jax.random.key(1234)
    u = jax.random.uniform(gkey, (NUM_SAMPLES,) + scores.shape,
                           dtype=scores.dtype, minval=1e-8, maxval=1.0 - 1e-8)
    g = -jnp.log(-jnp.log(u))
    p = scores[None, ...] + g
    out = pl.pallas_call(
        _dknn_block,
        grid=(NUM_SAMPLES, Q // QB),
        in_specs=[
            pl.BlockSpec((1, QB, N), lambda s, qb: (s, qb, 0)),
        ],
        out_specs=pl.BlockSpec((1, QB, N), lambda s, qb: (s, qb, 0)),
        out_shape=jax.ShapeDtypeStruct((NUM_SAMPLES, Q, N), query.dtype),
        compiler_params=pltpu.CompilerParams(
            dimension_semantics=("parallel", "parallel")),
    )(p)
    return out
